# baseline (speedup 1.0000x reference)
import math

import jax
import jax.numpy as jnp
from jax.experimental import pallas as pl
from jax.experimental.pallas import tpu as pltpu
from jax.experimental.pallas import tpu_sc as plsc

_H = 4


def _rowsum(a):
    c = [a[:, i * 8:(i + 1) * 8] for i in range(a.shape[-1] // 8)]
    r = c[0]
    for p in c[1:]:
        r = r + p
    while r.shape[-1] > 1:
        h = r.shape[-1] // 2
        r = r[:, :h] + r[:, h:]
    return r


def _ln(x, s, b, eps=1e-5):
    n = x.shape[-1]
    mu = _rowsum(x) / n
    var = _rowsum((x - mu) ** 2) / n
    return (x - mu) / jnp.sqrt(var + eps) * s + b


def _softmax(x):
    ex = jnp.exp(x - jnp.max(x, axis=-1, keepdims=True))
    return ex / _rowsum(ex)


def _rot_half(x):
    h = x.shape[-1] // 2
    return jnp.concatenate([-x[:, h:], x[:, :h]], axis=-1)


def _enc_vq_body(x_ref, cos_ref, sin_ref, Wq_ref, bq_ref, Wk_ref, bk_ref,
                 Wv_ref, bv_ref, Wo_ref, bo_ref, W1_ref, b1_ref, W2_ref,
                 b2_ref, l1s_ref, l1b_ref, l2s_ref, l2b_ref, cb_ref,
                 cbss_ref, idx_ref):
    f32 = jnp.float32
    x = x_ref[0]
    T, d = x.shape
    hd = d // _H
    cos = cos_ref[...]
    sin = sin_ref[...]
    L = Wq_ref.shape[0]
    scale = math.sqrt(float(hd))
    for l in range(L):
        q = jnp.dot(x, Wq_ref[l], preferred_element_type=f32) + bq_ref[l]
        k = jnp.dot(x, Wk_ref[l], preferred_element_type=f32) + bk_ref[l]
        v = jnp.dot(x, Wv_ref[l], preferred_element_type=f32) + bv_ref[l]
        parts = []
        for h in range(_H):
            sl = slice(h * hd, (h + 1) * hd)
            qh = q[:, sl]
            kh = k[:, sl]
            vh = v[:, sl]
            qh = qh * cos + _rot_half(qh) * sin
            kh = kh * cos + _rot_half(kh) * sin
            aw = jax.lax.dot_general(qh, kh, (((1,), (1,)), ((), ())),
                                     preferred_element_type=f32) / scale
            ex = jnp.exp(aw - jnp.max(aw, axis=-1, keepdims=True))
            s = _rowsum(ex)
            parts.append(jnp.dot(ex, vh, preferred_element_type=f32) / s)
        ao = jnp.concatenate(parts, axis=-1)
        ao = jnp.dot(ao, Wo_ref[l], preferred_element_type=f32) + bo_ref[l]
        x = _ln(x + ao, l1s_ref[l], l1b_ref[l])
        f = jnp.dot(
            jax.nn.gelu(jnp.dot(x, W1_ref[l], preferred_element_type=f32)
                        + b1_ref[l]),
            W2_ref[l], preferred_element_type=f32) + b2_ref[l]
        x = _ln(x + f, l2s_ref[l], l2b_ref[l])
    cb = cb_ref[...]
    K = cb.shape[0]
    rowss = _rowsum(x * x)
    d2 = (rowss
          - 2.0 * jax.lax.dot_general(x, cb, (((1,), (1,)), ((), ())),
                                      preferred_element_type=f32)
          + cbss_ref[...])
    m = jnp.min(d2, axis=1, keepdims=True)
    cols = jax.lax.broadcasted_iota(jnp.int32, d2.shape, 1)
    idx = jnp.min(jnp.where(d2 <= m, cols, K), axis=1)
    idx_ref[0, 0, :] = idx


def _encoder_vq(x, cos, sin, Wq, bq, Wk, bk, Wv, bv, Wo, bo, W1, b1, W2, b2,
                l1s, l1b, l2s, l2b, codebook, cbss, interpret=False):
    B, T, d = x.shape
    L = Wq.shape[0]
    const = lambda shape: pl.BlockSpec(shape, lambda b: (0,) * len(shape))
    in_specs = [
        pl.BlockSpec((1, T, d), lambda b: (b, 0, 0)),
        const(cos.shape), const(sin.shape),
        const(Wq.shape), const(bq.shape),
        const(Wk.shape), const(bk.shape),
        const(Wv.shape), const(bv.shape),
        const(Wo.shape), const(bo.shape),
        const(W1.shape), const(b1.shape),
        const(W2.shape), const(b2.shape),
        const(l1s.shape), const(l1b.shape),
        const(l2s.shape), const(l2b.shape),
        const(codebook.shape), const(cbss.shape),
    ]
    out = pl.pallas_call(
        _enc_vq_body,
        grid=(B,),
        in_specs=in_specs,
        out_specs=pl.BlockSpec((1, 1, T), lambda b: (b, 0, 0)),
        out_shape=jax.ShapeDtypeStruct((B, 1, T), jnp.int32),
        interpret=interpret,
    )(x, cos, sin, Wq, bq, Wk, bk, Wv, bv, Wo, bo, W1, b1, W2, b2,
      l1s, l1b, l2s, l2b, codebook, cbss)
    return out.reshape(B, T)


def _sc_gather(table, flat_idx):
    M = flat_idx.shape[1]
    W = 128
    mesh = plsc.VectorSubcoreMesh(core_axis_name="core",
                                  subcore_axis_name="subcore")

    @pl.kernel(out_type=jax.ShapeDtypeStruct((M, table.shape[1]), table.dtype),
               mesh=mesh)
    def k(tab_hbm, i_hbm, o_hbm):
        def body(i_vmem, o_vmem):
            pltpu.sync_copy(tab_hbm.at[i_vmem.at[0]], o_vmem)

        pltpu.emit_pipeline(
            body,
            grid=(M // W,),
            in_specs=[pl.BlockSpec((1, W), index_map=lambda i: (0, i))],
            out_specs=[pl.BlockSpec((W, table.shape[1]),
                                    index_map=lambda i: (i, 0))],
            core_axis_name=("core", "subcore"),
            dimension_semantics=(pltpu.PARALLEL,),
        )(i_hbm, o_hbm)

    return k(table, flat_idx)


def kernel(token_ids, token_embedding, Wq, bq, Wk, bk, Wv, bv, Wo, bo,
           W1, b1, W2, b2, ln1_s, ln1_b, ln2_s, ln2_b, codebook):
    B, T = token_ids.shape
    d = token_embedding.shape[1]
    L = Wq.shape[0]
    hd = d // _H
    inv_freq = 1.0 / (10000.0 ** (jnp.arange(0, hd, 2, dtype=jnp.float32) / hd))
    freqs = jnp.outer(jnp.arange(T, dtype=jnp.float32), inv_freq)
    emb = jnp.concatenate([freqs, freqs], axis=-1)
    cos, sin = jnp.cos(emb), jnp.sin(emb)
    cbss = jnp.sum(codebook ** 2, axis=1)[None, :]
    r = lambda a: a.reshape(L, 1, -1)

    flat_ids = token_ids.reshape(1, B * T).astype(jnp.int32)
    x = _sc_gather(token_embedding, flat_ids)
    idx = _encoder_vq(x.reshape(B, T, d), cos, sin, Wq, r(bq), Wk, r(bk),
                      Wv, r(bv), Wo, r(bo), W1, r(b1), W2, r(b2),
                      r(ln1_s), r(ln1_b), r(ln2_s), r(ln2_b),
                      codebook, cbss)
    qz = _sc_gather(codebook, idx.reshape(1, B * T))
    quant = qz.reshape(B, T, d)
    return quant, idx
